# Initial kernel scaffold; baseline (speedup 1.0000x reference)
import jax
import jax.numpy as jnp
from jax.experimental import pallas as pl


def kernel(x, edge_index, edge_weight, W):
    raise NotImplementedError("write your pallas kernel here")


# speedup vs baseline: 3.7562x; 3.7562x over previous
import functools

import jax
import jax.numpy as jnp
from jax import lax
from jax.experimental import pallas as pl
from jax.experimental.pallas import tpu as pltpu
from jax.experimental.pallas import tpu_sc as plsc

N = 10000
E = 320000
D = 128
NC = 2
NS = 16
L = 16
NW = NC * NS
EPW = E // NW
CH = 80
NCHUNK = EPW // CH
ROWS_PER_TILE = N // NS


def _matmul_body(x_ref, w_ref, o_ref):
    o_ref[...] = jnp.dot(x_ref[...], w_ref[...],
                         preferred_element_type=jnp.float32)


def _edge_body(ps_hbm, src_hbm, dst_hbm, w_hbm, out_hbm,
               acc, rows, src_v, dst_v, w_v, sem):
    cid = lax.axis_index("c")
    sid = lax.axis_index("s")
    wid = sid * NC + cid
    base = wid * EPW

    def _zero_row(i, _):
        for j in range(D // L):
            rows[i, pl.ds(j * L, L)] = jnp.zeros((L,), jnp.float32)
        return 0
    lax.fori_loop(0, CH, _zero_row, 0)
    r0 = sid * ROWS_PER_TILE
    off = 0
    while off < ROWS_PER_TILE:
        sz = min(CH, ROWS_PER_TILE - off)
        pltpu.sync_copy(rows.at[pl.ds(0, sz)], acc.at[pl.ds(r0 + off, sz)])
        off += sz
    plsc.subcore_barrier()

    def _chunk(c, _):
        eoff = pl.multiple_of(base + c * CH, 8)
        pltpu.sync_copy(src_hbm.at[pl.ds(eoff, CH)], src_v)
        pltpu.sync_copy(dst_hbm.at[pl.ds(eoff, CH)], dst_v)
        pltpu.sync_copy(w_hbm.at[pl.ds(eoff, CH)], w_v)
        pltpu.async_copy(ps_hbm.at[src_v], rows, sem).wait()

        def _edge(e, _):
            wv = plsc.load_gather(w_v, [jnp.full((L,), e, jnp.int32)])
            for j in range(D // L):
                sl = (e, pl.ds(j * L, L))
                rows[sl] = rows[sl] * wv
            return 0
        lax.fori_loop(0, CH, _edge, 0)

        pltpu.sync_copy(rows, acc.at[dst_v], add=True)
        return 0

    lax.fori_loop(0, NCHUNK, _chunk, 0)
    plsc.subcore_barrier()

    @pl.when(sid == 0)
    def _():
        pltpu.sync_copy(acc, out_hbm.at[cid])


def _norm_body(p_ref, o_ref):
    s = p_ref[0] + p_ref[1]
    mean = jnp.mean(s, axis=0, keepdims=True)
    d = s - mean
    var = jnp.mean(d * d, axis=0, keepdims=True)
    o_ref[...] = jnp.maximum(d * lax.rsqrt(var + 0.001), 0.0)


@jax.jit
def kernel(x, edge_index, edge_weight, W):
    pre_sup = pl.pallas_call(
        _matmul_body,
        out_shape=jax.ShapeDtypeStruct((N, D), jnp.float32),
    )(x, W)

    mesh = plsc.VectorSubcoreMesh(core_axis_name="c", subcore_axis_name="s",
                                  num_cores=NC, num_subcores=NS)
    partials = pl.kernel(
        _edge_body,
        out_type=jax.ShapeDtypeStruct((NC, N, D), jnp.float32),
        mesh=mesh,
        compiler_params=pltpu.CompilerParams(needs_layout_passes=False),
        scratch_types=[
            pltpu.MemorySpace.VMEM_SHARED((N, D), jnp.float32),
            pltpu.VMEM((CH, D), jnp.float32),
            pltpu.VMEM((CH,), jnp.int32),
            pltpu.VMEM((CH,), jnp.int32),
            pltpu.VMEM((CH,), jnp.float32),
            pltpu.SemaphoreType.DMA,
        ],
    )(pre_sup, edge_index[0], edge_index[1], edge_weight)

    return pl.pallas_call(
        _norm_body,
        out_shape=jax.ShapeDtypeStruct((N, D), jnp.float32),
    )(partials)


# speedup vs baseline: 11.5873x; 3.0849x over previous
import jax
import jax.numpy as jnp
from jax import lax
from jax.experimental import pallas as pl
from jax.experimental.pallas import tpu as pltpu
from jax.experimental.pallas import tpu_sc as plsc

N = 10000
E = 320000
D = 128
NC = 2
NS = 16
L = 16
NW = NC * NS
CH = 80
EPW = E // NW
NCHUNK = EPW // CH
NB = 4
NI = 8
ROWS_PER_TILE = 624
REM_ROWS = N - NS * ROWS_PER_TILE


def _matmul_body(x_ref, w_ref, o_ref):
    o_ref[...] = jnp.dot(x_ref[...], w_ref[...],
                         preferred_element_type=jnp.float32)


def _edge_body(ps_hbm, src_hbm, dst_hbm, w_hbm, out_hbm, acc,
               rows0, rows1, rows2, rows3,
               src0, src1, src2, src3, src4, src5, src6, src7,
               dst0, dst1, dst2, dst3, dst4, dst5, dst6, dst7,
               w0, w1, w2, w3, w4, w5, w6, w7,
               si0, si1, si2, si3, si4, si5, si6, si7,
               sg0, sg1, sg2, sg3, ss0, ss1, ss2, ss3):
    rows = (rows0, rows1, rows2, rows3)
    srcb = (src0, src1, src2, src3, src4, src5, src6, src7)
    dstb = (dst0, dst1, dst2, dst3, dst4, dst5, dst6, dst7)
    wb = (w0, w1, w2, w3, w4, w5, w6, w7)
    si = (si0, si1, si2, si3, si4, si5, si6, si7)
    sg = (sg0, sg1, sg2, sg3)
    ss = (ss0, ss1, ss2, ss3)
    cid = lax.axis_index("c")
    sid = lax.axis_index("s")
    wid = sid * NC + cid
    ebase = wid * EPW

    def _idx_issue(c, j):
        eoff = pl.multiple_of(ebase + c * CH, 8)
        pltpu.async_copy(src_hbm.at[pl.ds(eoff, CH)], srcb[j], si[j])
        pltpu.async_copy(dst_hbm.at[pl.ds(eoff, CH)], dstb[j], si[j])
        pltpu.async_copy(w_hbm.at[pl.ds(eoff, CH)], wb[j], si[j])

    def _idx_wait(c, j):
        eoff = pl.multiple_of(ebase + c * CH, 8)
        pltpu.make_async_copy(src_hbm.at[pl.ds(eoff, CH)], srcb[j], si[j]).wait()
        pltpu.make_async_copy(dst_hbm.at[pl.ds(eoff, CH)], dstb[j], si[j]).wait()
        pltpu.make_async_copy(w_hbm.at[pl.ds(eoff, CH)], wb[j], si[j]).wait()

    def _gather_issue(k, j):
        pltpu.async_copy(ps_hbm.at[srcb[j]], rows[k], sg[k])

    def _gather_wait(k, j):
        pltpu.make_async_copy(ps_hbm.at[srcb[j]], rows[k], sg[k]).wait()

    def _scatter_issue(k, j):
        pltpu.async_copy(rows[k], acc.at[dstb[j]], ss[k], add=True)

    def _scatter_wait(k, j):
        pltpu.make_async_copy(rows[k], acc.at[dstb[j]], ss[k]).wait()

    def _scale(k, j):
        rk = rows[k]
        wk = wb[j]

        @plsc.parallel_loop(0, CH, unroll=4)
        def _(e):
            wvl = plsc.load_gather(wk, [jnp.full((L,), e, jnp.int32)])
            for jj in range(D // L):
                sl = (e, pl.ds(jj * L, L))
                rk[sl] = rk[sl] * wvl

    for j in range(4):
        _idx_issue(j, j)

    def _zero_row(i, _):
        for jj in range(D // L):
            rows0[i, pl.ds(jj * L, L)] = jnp.zeros((L,), jnp.float32)
        return 0
    lax.fori_loop(0, CH, _zero_row, 0)
    r0 = sid * ROWS_PER_TILE
    off = 0
    while off < ROWS_PER_TILE:
        sz = min(CH, ROWS_PER_TILE - off)
        pltpu.sync_copy(rows0.at[pl.ds(0, sz)], acc.at[pl.ds(r0 + off, sz)])
        off += sz

    @pl.when(sid == 0)
    def _():
        pltpu.sync_copy(rows0.at[pl.ds(0, REM_ROWS)],
                        acc.at[pl.ds(NS * ROWS_PER_TILE, REM_ROWS)])
    plsc.subcore_barrier()

    _idx_wait(0, 0)
    _gather_issue(0, 0)
    _idx_wait(1, 1)
    _gather_issue(1, 1)

    def _step(c, k, j):
        _gather_wait(k, j)

        @pl.when(c >= 2)
        def _():
            _scatter_wait((k + 2) % NB, (j + 6) % NI)

        @pl.when(c + 2 <= NCHUNK - 1)
        def _():
            _idx_wait(c + 2, (j + 2) % NI)
            _gather_issue((k + 2) % NB, (j + 2) % NI)

        @pl.when(c + 4 <= NCHUNK - 1)
        def _():
            _idx_issue(c + 4, (j + 4) % NI)

        _scale(k, j)
        _scatter_issue(k, j)

    nloop = (NCHUNK // NI) * NI

    @pl.loop(0, nloop, step=NI)
    def _(cbase):
        for u in range(NI):
            _step(cbase + u, u % NB, u)

    for c in range(nloop, NCHUNK):
        _step(c, c % NB, c % NI)
    _scatter_wait((NCHUNK - 2) % NB, (NCHUNK - 2) % NI)
    _scatter_wait((NCHUNK - 1) % NB, (NCHUNK - 1) % NI)

    plsc.subcore_barrier()
    pltpu.sync_copy(acc.at[pl.ds(r0, ROWS_PER_TILE)],
                    out_hbm.at[cid, pl.ds(r0, ROWS_PER_TILE)])

    @pl.when(sid == 0)
    def _():
        pltpu.sync_copy(acc.at[pl.ds(NS * ROWS_PER_TILE, REM_ROWS)],
                        out_hbm.at[cid, pl.ds(NS * ROWS_PER_TILE, REM_ROWS)])


def _norm_body(p_ref, o_ref):
    s = p_ref[0] + p_ref[1]
    mean = jnp.mean(s, axis=0, keepdims=True)
    d = s - mean
    var = jnp.mean(d * d, axis=0, keepdims=True)
    o_ref[...] = jnp.maximum(d * lax.rsqrt(var + 0.001), 0.0)


@jax.jit
def kernel(x, edge_index, edge_weight, W):
    pre_sup = pl.pallas_call(
        _matmul_body,
        out_shape=jax.ShapeDtypeStruct((N, D), jnp.float32),
    )(x, W)

    mesh = plsc.VectorSubcoreMesh(core_axis_name="c", subcore_axis_name="s",
                                  num_cores=NC, num_subcores=NS)
    partials = pl.kernel(
        _edge_body,
        out_type=jax.ShapeDtypeStruct((NC, N, D), jnp.float32),
        mesh=mesh,
        compiler_params=pltpu.CompilerParams(needs_layout_passes=False),
        scratch_types=[
            pltpu.MemorySpace.VMEM_SHARED((N, D), jnp.float32),
        ] + [pltpu.VMEM((CH, D), jnp.float32)] * NB
          + [pltpu.VMEM((CH,), jnp.int32)] * (2 * NI)
          + [pltpu.VMEM((CH,), jnp.float32)] * NI
          + [pltpu.SemaphoreType.DMA] * (NI + 2 * NB),
    )(pre_sup, edge_index[0], edge_index[1], edge_weight)

    return pl.pallas_call(
        _norm_body,
        out_shape=jax.ShapeDtypeStruct((N, D), jnp.float32),
    )(partials)


# speedup vs baseline: 11.9601x; 1.0322x over previous
import jax
import jax.numpy as jnp
from jax import lax
from jax.experimental import pallas as pl
from jax.experimental.pallas import tpu as pltpu
from jax.experimental.pallas import tpu_sc as plsc

N = 10000
E = 320000
D = 128
NC = 2
NS = 16
L = 16
NW = NC * NS
CH = 80
EPW = E // NW
NCHUNK = EPW // CH
NB = 4
NI = 8
ROWS_PER_TILE = 624
REM_ROWS = N - NS * ROWS_PER_TILE


def _edge_body(ps_hbm, src_hbm, dst_hbm, w_hbm, out_hbm, acc,
               rows0, rows1, rows2, rows3,
               src0, src1, src2, src3, src4, src5, src6, src7,
               dst0, dst1, dst2, dst3, dst4, dst5, dst6, dst7,
               w0, w1, w2, w3, w4, w5, w6, w7,
               si0, si1, si2, si3, si4, si5, si6, si7,
               sg0, sg1, sg2, sg3, ss0, ss1, ss2, ss3):
    rows = (rows0, rows1, rows2, rows3)
    srcb = (src0, src1, src2, src3, src4, src5, src6, src7)
    dstb = (dst0, dst1, dst2, dst3, dst4, dst5, dst6, dst7)
    wb = (w0, w1, w2, w3, w4, w5, w6, w7)
    si = (si0, si1, si2, si3, si4, si5, si6, si7)
    sg = (sg0, sg1, sg2, sg3)
    ss = (ss0, ss1, ss2, ss3)
    cid = lax.axis_index("c")
    sid = lax.axis_index("s")
    wid = sid * NC + cid
    ebase = wid * EPW

    def _idx_issue(c, j):
        eoff = pl.multiple_of(ebase + c * CH, 8)
        pltpu.async_copy(src_hbm.at[pl.ds(eoff, CH)], srcb[j], si[j])
        pltpu.async_copy(dst_hbm.at[pl.ds(eoff, CH)], dstb[j], si[j])
        pltpu.async_copy(w_hbm.at[pl.ds(eoff, CH)], wb[j], si[j])

    def _idx_wait(c, j):
        eoff = pl.multiple_of(ebase + c * CH, 8)
        pltpu.make_async_copy(src_hbm.at[pl.ds(eoff, CH)], srcb[j], si[j]).wait()
        pltpu.make_async_copy(dst_hbm.at[pl.ds(eoff, CH)], dstb[j], si[j]).wait()
        pltpu.make_async_copy(w_hbm.at[pl.ds(eoff, CH)], wb[j], si[j]).wait()

    def _gather_issue(k, j):
        pltpu.async_copy(ps_hbm.at[srcb[j]], rows[k], sg[k])

    def _gather_wait(k, j):
        pltpu.make_async_copy(ps_hbm.at[srcb[j]], rows[k], sg[k]).wait()

    def _scatter_issue(k, j):
        pltpu.async_copy(rows[k], acc.at[dstb[j]], ss[k], add=True)

    def _scatter_wait(k, j):
        pltpu.make_async_copy(rows[k], acc.at[dstb[j]], ss[k]).wait()

    def _scale(k, j):
        rk = rows[k]
        wk = wb[j]

        @plsc.parallel_loop(0, CH, unroll=4)
        def _(e):
            wvl = plsc.load_gather(wk, [jnp.full((L,), e, jnp.int32)])
            for jj in range(D // L):
                sl = (e, pl.ds(jj * L, L))
                rk[sl] = rk[sl] * wvl

    for j in range(4):
        _idx_issue(j, j)

    def _zero_row(i, _):
        for jj in range(D // L):
            rows0[i, pl.ds(jj * L, L)] = jnp.zeros((L,), jnp.float32)
        return 0
    lax.fori_loop(0, CH, _zero_row, 0)
    r0 = sid * ROWS_PER_TILE
    off = 0
    while off < ROWS_PER_TILE:
        sz = min(CH, ROWS_PER_TILE - off)
        pltpu.sync_copy(rows0.at[pl.ds(0, sz)], acc.at[pl.ds(r0 + off, sz)])
        off += sz

    @pl.when(sid == 0)
    def _():
        pltpu.sync_copy(rows0.at[pl.ds(0, REM_ROWS)],
                        acc.at[pl.ds(NS * ROWS_PER_TILE, REM_ROWS)])
    plsc.subcore_barrier()

    _idx_wait(0, 0)
    _gather_issue(0, 0)
    _idx_wait(1, 1)
    _gather_issue(1, 1)

    def _step(c, k, j):
        _gather_wait(k, j)

        @pl.when(c >= 2)
        def _():
            _scatter_wait((k + 2) % NB, (j + 6) % NI)

        @pl.when(c + 2 <= NCHUNK - 1)
        def _():
            _idx_wait(c + 2, (j + 2) % NI)
            _gather_issue((k + 2) % NB, (j + 2) % NI)

        @pl.when(c + 4 <= NCHUNK - 1)
        def _():
            _idx_issue(c + 4, (j + 4) % NI)

        _scale(k, j)
        _scatter_issue(k, j)

    nloop = (NCHUNK // NI) * NI

    @pl.loop(0, nloop, step=NI)
    def _(cbase):
        for u in range(NI):
            _step(cbase + u, u % NB, u)

    for c in range(nloop, NCHUNK):
        _step(c, c % NB, c % NI)
    _scatter_wait((NCHUNK - 2) % NB, (NCHUNK - 2) % NI)
    _scatter_wait((NCHUNK - 1) % NB, (NCHUNK - 1) % NI)

    plsc.subcore_barrier()
    pltpu.sync_copy(acc.at[pl.ds(r0, ROWS_PER_TILE)],
                    out_hbm.at[cid, pl.ds(r0, ROWS_PER_TILE)])

    @pl.when(sid == 0)
    def _():
        pltpu.sync_copy(acc.at[pl.ds(NS * ROWS_PER_TILE, REM_ROWS)],
                        out_hbm.at[cid, pl.ds(NS * ROWS_PER_TILE, REM_ROWS)])


def _mm_norm_body(p_ref, w_ref, o_ref):
    s = jnp.dot(p_ref[0] + p_ref[1], w_ref[...],
                preferred_element_type=jnp.float32)
    mean = jnp.mean(s, axis=0, keepdims=True)
    d = s - mean
    var = jnp.mean(d * d, axis=0, keepdims=True)
    o_ref[...] = jnp.maximum(d * lax.rsqrt(var + 0.001), 0.0)


@jax.jit
def kernel(x, edge_index, edge_weight, W):
    mesh = plsc.VectorSubcoreMesh(core_axis_name="c", subcore_axis_name="s",
                                  num_cores=NC, num_subcores=NS)
    partials = pl.kernel(
        _edge_body,
        out_type=jax.ShapeDtypeStruct((NC, N, D), jnp.float32),
        mesh=mesh,
        compiler_params=pltpu.CompilerParams(needs_layout_passes=False),
        scratch_types=[
            pltpu.MemorySpace.VMEM_SHARED((N, D), jnp.float32),
        ] + [pltpu.VMEM((CH, D), jnp.float32)] * NB
          + [pltpu.VMEM((CH,), jnp.int32)] * (2 * NI)
          + [pltpu.VMEM((CH,), jnp.float32)] * NI
          + [pltpu.SemaphoreType.DMA] * (NI + 2 * NB),
    )(x, edge_index[0], edge_index[1], edge_weight)

    return pl.pallas_call(
        _mm_norm_body,
        out_shape=jax.ShapeDtypeStruct((N, D), jnp.float32),
    )(partials, W)


# speedup vs baseline: 12.2953x; 1.0280x over previous
import jax
import jax.numpy as jnp
from jax import lax
from jax.experimental import pallas as pl
from jax.experimental.pallas import tpu as pltpu
from jax.experimental.pallas import tpu_sc as plsc

N = 10000
E = 320000
D = 128
NC = 2
NS = 16
L = 16
NW = NC * NS
CH = 40
EPW = E // NW
NCHUNK = EPW // CH
NB = 6
NI = 12
GAH = NB - 2
IAH = GAH + 2
NLOOP = (NCHUNK // NI) * NI
ROWS_PER_TILE = 624
REM_ROWS = N - NS * ROWS_PER_TILE


def _edge_body(x_hbm, src_hbm, dst_hbm, w_hbm, out_hbm, acc, *rest):
    rows = rest[0:NB]
    srcb = rest[NB:NB + NI]
    dstb = rest[NB + NI:NB + 2 * NI]
    wb = rest[NB + 2 * NI:NB + 3 * NI]
    si = rest[NB + 3 * NI:NB + 4 * NI]
    sg = rest[NB + 4 * NI:NB + 4 * NI + NB]
    ss = rest[NB + 4 * NI + NB:NB + 4 * NI + 2 * NB]
    cid = lax.axis_index("c")
    sid = lax.axis_index("s")
    wid = sid * NC + cid
    ebase = wid * EPW

    def _idx_issue(c, j):
        eoff = pl.multiple_of(ebase + c * CH, 8)
        pltpu.async_copy(src_hbm.at[pl.ds(eoff, CH)], srcb[j], si[j])
        pltpu.async_copy(dst_hbm.at[pl.ds(eoff, CH)], dstb[j], si[j])
        pltpu.async_copy(w_hbm.at[pl.ds(eoff, CH)], wb[j], si[j])

    def _idx_wait(c, j):
        eoff = pl.multiple_of(ebase + c * CH, 8)
        pltpu.make_async_copy(src_hbm.at[pl.ds(eoff, CH)], srcb[j],
                              si[j]).wait()
        pltpu.make_async_copy(dst_hbm.at[pl.ds(eoff, CH)], dstb[j],
                              si[j]).wait()
        pltpu.make_async_copy(w_hbm.at[pl.ds(eoff, CH)], wb[j],
                              si[j]).wait()

    def _gather_issue(k, j):
        pltpu.async_copy(x_hbm.at[srcb[j]], rows[k], sg[k])

    def _gather_wait(k, j):
        pltpu.make_async_copy(x_hbm.at[srcb[j]], rows[k], sg[k]).wait()

    def _scatter_issue(k, j):
        pltpu.async_copy(rows[k], acc.at[dstb[j]], ss[k], add=True)

    def _scatter_wait(k, j):
        pltpu.make_async_copy(rows[k], acc.at[dstb[j]], ss[k]).wait()

    def _scale(k, j):
        rk = rows[k]
        wk = wb[j]

        @plsc.parallel_loop(0, CH, unroll=4)
        def _(e):
            wvl = plsc.load_gather(wk, [jnp.full((L,), e, jnp.int32)])
            for jj in range(D // L):
                sl = (e, pl.ds(jj * L, L))
                rk[sl] = rk[sl] * wvl

    for c in range(IAH):
        _idx_issue(c, c)

    def _zero_row(i, _):
        for jj in range(D // L):
            rows[0][i, pl.ds(jj * L, L)] = jnp.zeros((L,), jnp.float32)
        return 0
    lax.fori_loop(0, CH, _zero_row, 0)
    r0 = sid * ROWS_PER_TILE
    off = 0
    while off < ROWS_PER_TILE:
        sz = min(CH, ROWS_PER_TILE - off)
        pltpu.sync_copy(rows[0].at[pl.ds(0, sz)], acc.at[pl.ds(r0 + off, sz)])
        off += sz

    @pl.when(sid == 0)
    def _():
        pltpu.sync_copy(rows[0].at[pl.ds(0, REM_ROWS)],
                        acc.at[pl.ds(NS * ROWS_PER_TILE, REM_ROWS)])
    plsc.subcore_barrier()

    for c in range(GAH):
        _idx_wait(c, c)
        _gather_issue(c, c)

    def _step(c, k, j):
        _gather_wait(k, j)

        @pl.when(c >= 2)
        def _():
            _scatter_wait((k + NB - 2) % NB, (j + NI - 2) % NI)

        @pl.when(c + GAH <= NCHUNK - 1)
        def _():
            _idx_wait(c + GAH, (j + GAH) % NI)
            _gather_issue((k + GAH) % NB, (j + GAH) % NI)

        @pl.when(c + IAH <= NCHUNK - 1)
        def _():
            _idx_issue(c + IAH, (j + IAH) % NI)

        _scale(k, j)
        _scatter_issue(k, j)

    @pl.loop(0, NLOOP, step=NI)
    def _(cbase):
        for u in range(NI):
            _step(cbase + u, u % NB, u)

    for c in range(NLOOP, NCHUNK):
        _step(c, c % NB, c % NI)
    _scatter_wait((NCHUNK - 2) % NB, (NCHUNK - 2) % NI)
    _scatter_wait((NCHUNK - 1) % NB, (NCHUNK - 1) % NI)

    plsc.subcore_barrier()
    pltpu.sync_copy(acc.at[pl.ds(r0, ROWS_PER_TILE)],
                    out_hbm.at[cid, pl.ds(r0, ROWS_PER_TILE)])

    @pl.when(sid == 0)
    def _():
        pltpu.sync_copy(acc.at[pl.ds(NS * ROWS_PER_TILE, REM_ROWS)],
                        out_hbm.at[cid, pl.ds(NS * ROWS_PER_TILE, REM_ROWS)])


def _mm_norm_body(p_ref, w_ref, o_ref):
    s = jnp.dot(p_ref[0] + p_ref[1], w_ref[...],
                preferred_element_type=jnp.float32)
    mean = jnp.mean(s, axis=0, keepdims=True)
    d = s - mean
    var = jnp.mean(d * d, axis=0, keepdims=True)
    o_ref[...] = jnp.maximum(d * lax.rsqrt(var + 0.001), 0.0)


@jax.jit
def kernel(x, edge_index, edge_weight, W):
    mesh = plsc.VectorSubcoreMesh(core_axis_name="c", subcore_axis_name="s",
                                  num_cores=NC, num_subcores=NS)
    partials = pl.kernel(
        _edge_body,
        out_type=jax.ShapeDtypeStruct((NC, N, D), jnp.float32),
        mesh=mesh,
        compiler_params=pltpu.CompilerParams(needs_layout_passes=False),
        scratch_types=[
            pltpu.MemorySpace.VMEM_SHARED((N, D), jnp.float32),
        ] + [pltpu.VMEM((CH, D), jnp.float32)] * NB
          + [pltpu.VMEM((CH,), jnp.int32)] * NI
          + [pltpu.VMEM((CH,), jnp.int32)] * NI
          + [pltpu.VMEM((CH,), jnp.float32)] * NI
          + [pltpu.SemaphoreType.DMA] * (NI + 2 * NB),
    )(x, edge_index[0], edge_index[1], edge_weight)

    return pl.pallas_call(
        _mm_norm_body,
        out_shape=jax.ShapeDtypeStruct((N, D), jnp.float32),
    )(partials, W)


# speedup vs baseline: 12.3586x; 1.0052x over previous
import jax
import jax.numpy as jnp
from jax import lax
from jax.experimental import pallas as pl
from jax.experimental.pallas import tpu as pltpu
from jax.experimental.pallas import tpu_sc as plsc

N = 10000
E = 320000
D = 128
NC = 2
NS = 16
L = 16
NW = NC * NS
CH = 40
EPW = E // NW
NCHUNK = EPW // CH
NB = 6
NI = 12
GAH = NB - 2
IAH = GAH + 2
NLOOP = (NCHUNK // NI) * NI
ROWS_PER_TILE = 624
REM_ROWS = N - NS * ROWS_PER_TILE


def _edge_body(x_hbm, src_hbm, dst_hbm, w_hbm, out_hbm, acc, *rest):
    rows = rest[0:NB]
    srcb = rest[NB:NB + NI]
    dstb = rest[NB + NI:NB + 2 * NI]
    wb = rest[NB + 2 * NI:NB + 3 * NI]
    si = rest[NB + 3 * NI:NB + 4 * NI]
    sg = rest[NB + 4 * NI:NB + 4 * NI + NB]
    ss = rest[NB + 4 * NI + NB:NB + 4 * NI + 2 * NB]
    cid = lax.axis_index("c")
    sid = lax.axis_index("s")
    wid = sid * NC + cid
    ebase = wid * EPW

    def _idx_issue(c, j):
        eoff = pl.multiple_of(ebase + c * CH, 8)
        pltpu.async_copy(src_hbm.at[pl.ds(eoff, CH)], srcb[j], si[j])
        pltpu.async_copy(dst_hbm.at[pl.ds(eoff, CH)], dstb[j], si[j])
        pltpu.async_copy(w_hbm.at[pl.ds(eoff, CH)], wb[j], si[j])

    def _idx_wait(c, j):
        eoff = pl.multiple_of(ebase + c * CH, 8)
        pltpu.make_async_copy(src_hbm.at[pl.ds(eoff, CH)], srcb[j],
                              si[j]).wait()
        pltpu.make_async_copy(dst_hbm.at[pl.ds(eoff, CH)], dstb[j],
                              si[j]).wait()
        pltpu.make_async_copy(w_hbm.at[pl.ds(eoff, CH)], wb[j],
                              si[j]).wait()

    def _gather_issue(k, j):
        pltpu.async_copy(x_hbm.at[srcb[j]], rows[k], sg[k])

    def _gather_wait(k, j):
        pltpu.make_async_copy(x_hbm.at[srcb[j]], rows[k], sg[k]).wait()

    def _scatter_issue(k, j):
        pltpu.async_copy(rows[k], acc.at[dstb[j]], ss[k], add=True)

    def _scatter_wait(k, j):
        pltpu.make_async_copy(rows[k], acc.at[dstb[j]], ss[k]).wait()

    def _scale(k, j):
        rk = rows[k]
        wk = wb[j]

        @plsc.parallel_loop(0, CH, unroll=4)
        def _(e):
            wvl = plsc.load_gather(wk, [jnp.full((L,), e, jnp.int32)])
            for jj in range(D // L):
                sl = (e, pl.ds(jj * L, L))
                rk[sl] = rk[sl] * wvl

    for c in range(IAH):
        _idx_issue(c, c)

    def _zero_row(i, _):
        for jj in range(D // L):
            rows[0][i, pl.ds(jj * L, L)] = jnp.zeros((L,), jnp.float32)
        return 0
    lax.fori_loop(0, CH, _zero_row, 0)
    r0 = sid * ROWS_PER_TILE
    zcopies = []
    off = 0
    while off < ROWS_PER_TILE:
        sz = min(CH, ROWS_PER_TILE - off)
        zcopies.append((off, sz))
        off += sz
    for off, sz in zcopies:
        pltpu.async_copy(rows[0].at[pl.ds(0, sz)],
                         acc.at[pl.ds(r0 + off, sz)], sg[0])

    @pl.when(sid == 0)
    def _():
        pltpu.async_copy(rows[0].at[pl.ds(0, REM_ROWS)],
                         acc.at[pl.ds(NS * ROWS_PER_TILE, REM_ROWS)], sg[0])
    for off, sz in zcopies:
        pltpu.make_async_copy(rows[0].at[pl.ds(0, sz)],
                              acc.at[pl.ds(r0 + off, sz)], sg[0]).wait()

    @pl.when(sid == 0)
    def _():
        pltpu.make_async_copy(rows[0].at[pl.ds(0, REM_ROWS)],
                              acc.at[pl.ds(NS * ROWS_PER_TILE, REM_ROWS)],
                              sg[0]).wait()
    plsc.subcore_barrier()

    for c in range(GAH):
        _idx_wait(c, c)
        _gather_issue(c, c)

    def _step(c, k, j):
        _gather_wait(k, j)

        @pl.when(c >= 2)
        def _():
            _scatter_wait((k + NB - 2) % NB, (j + NI - 2) % NI)

        @pl.when(c + GAH <= NCHUNK - 1)
        def _():
            _idx_wait(c + GAH, (j + GAH) % NI)
            _gather_issue((k + GAH) % NB, (j + GAH) % NI)

        @pl.when(c + IAH <= NCHUNK - 1)
        def _():
            _idx_issue(c + IAH, (j + IAH) % NI)

        _scale(k, j)
        _scatter_issue(k, j)

    @pl.loop(0, NLOOP, step=NI)
    def _(cbase):
        for u in range(NI):
            _step(cbase + u, u % NB, u)

    for c in range(NLOOP, NCHUNK):
        _step(c, c % NB, c % NI)
    _scatter_wait((NCHUNK - 2) % NB, (NCHUNK - 2) % NI)
    _scatter_wait((NCHUNK - 1) % NB, (NCHUNK - 1) % NI)

    plsc.subcore_barrier()
    pltpu.async_copy(acc.at[pl.ds(r0, ROWS_PER_TILE)],
                     out_hbm.at[cid, pl.ds(r0, ROWS_PER_TILE)], sg[0])

    @pl.when(sid == 0)
    def _():
        pltpu.async_copy(acc.at[pl.ds(NS * ROWS_PER_TILE, REM_ROWS)],
                         out_hbm.at[cid, pl.ds(NS * ROWS_PER_TILE, REM_ROWS)],
                         sg[1])
    pltpu.make_async_copy(acc.at[pl.ds(r0, ROWS_PER_TILE)],
                          out_hbm.at[cid, pl.ds(r0, ROWS_PER_TILE)],
                          sg[0]).wait()

    @pl.when(sid == 0)
    def _():
        pltpu.make_async_copy(acc.at[pl.ds(NS * ROWS_PER_TILE, REM_ROWS)],
                              out_hbm.at[cid, pl.ds(NS * ROWS_PER_TILE, REM_ROWS)],
                              sg[1]).wait()


def _mm_norm_body(p_ref, w_ref, o_ref):
    s = jnp.dot(p_ref[0] + p_ref[1], w_ref[...],
                preferred_element_type=jnp.float32)
    mean = jnp.mean(s, axis=0, keepdims=True)
    d = s - mean
    var = jnp.mean(d * d, axis=0, keepdims=True)
    o_ref[...] = jnp.maximum(d * lax.rsqrt(var + 0.001), 0.0)


@jax.jit
def kernel(x, edge_index, edge_weight, W):
    mesh = plsc.VectorSubcoreMesh(core_axis_name="c", subcore_axis_name="s",
                                  num_cores=NC, num_subcores=NS)
    partials = pl.kernel(
        _edge_body,
        out_type=jax.ShapeDtypeStruct((NC, N, D), jnp.float32),
        mesh=mesh,
        compiler_params=pltpu.CompilerParams(needs_layout_passes=False),
        scratch_types=[
            pltpu.MemorySpace.VMEM_SHARED((N, D), jnp.float32),
        ] + [pltpu.VMEM((CH, D), jnp.float32)] * NB
          + [pltpu.VMEM((CH,), jnp.int32)] * NI
          + [pltpu.VMEM((CH,), jnp.int32)] * NI
          + [pltpu.VMEM((CH,), jnp.float32)] * NI
          + [pltpu.SemaphoreType.DMA] * (NI + 2 * NB),
    )(x, edge_index[0], edge_index[1], edge_weight)

    return pl.pallas_call(
        _mm_norm_body,
        out_shape=jax.ShapeDtypeStruct((N, D), jnp.float32),
    )(partials, W)
